# baseline (speedup 1.0000x reference)
import jax
import jax.numpy as jnp
from jax import lax
from jax.experimental import pallas as pl
from jax.experimental.pallas import tpu as pltpu
from jax.experimental.pallas import tpu_sc as plsc

D_DIM = 32
NI = 4096
NJ = 200
NUM_CORES = 2
NUM_SUBCORES = 16
JG = 8
IG = 4
JPW = NJ // JG
IPW = NI // IG
CH = 512
HALVES = IPW // CH
KT = D_DIM // 8
ITL = CH // 128


def _emb_body(w_hbm, idx_hbm, out_hbm, idx_v, g_v, s_v, gsem, ssem):
    c = lax.axis_index("c")
    s = lax.axis_index("s")
    w = s * NUM_CORES + c
    jg = w // IG
    ig = w % IG
    j0 = jg * JPW
    it_base = ig * (IPW // 128)

    pltpu.sync_copy(idx_hbm.at[pl.ds(j0, JPW), pl.ds(ig * IPW, IPW)], idx_v)

    iota = lax.iota(jnp.int32, 16)

    def gather_start(jl, half, b):
        pltpu.async_copy(w_hbm.at[idx_v.at[jl, pl.ds(half * CH, CH)]],
                         g_v.at[b], gsem.at[b])

    def gather_wait(jl, half, b):
        pltpu.make_async_copy(w_hbm.at[idx_v.at[jl, pl.ds(half * CH, CH)]],
                              g_v.at[b], gsem.at[b]).wait()

    def transpose(b):
        gv = g_v.at[b]
        for kt in range(KT):
            for itl in range(ITL):
                def krbody(kr, _, kt=kt, itl=itl):
                    cidx = jnp.full((16,), 8 * kt, jnp.int32) + kr
                    for v in range(8):
                        ridx = iota + (itl * 128 + 16 * v)
                        vec = plsc.load_gather(gv, [ridx, cidx])
                        s_v[b, kt, itl, kr, pl.ds(16 * v, 16)] = vec
                    return 0
                lax.fori_loop(0, 8, krbody, 0)

    def store_start(jl, half, b):
        for kt in range(KT):
            pltpu.async_copy(
                s_v.at[b, kt],
                out_hbm.at[j0 + jl, kt, pl.ds(it_base + half * ITL, ITL)],
                ssem.at[b])

    def store_wait(jl, half, b):
        for kt in range(KT):
            pltpu.make_async_copy(
                s_v.at[b, kt],
                out_hbm.at[j0 + jl, kt, pl.ds(it_base + half * ITL, ITL)],
                ssem.at[b]).wait()

    gather_start(0, 0, 0)
    gather_start(0, 1, 1)

    def outer(t, _):
        for half, b in ((0, 0), (1, 1)):
            gather_wait(t, half, b)

            @pl.when(t >= 1)
            def _(half=half, b=b):
                store_wait(t - 1, half, b)

            transpose(b)
            store_start(t, half, b)

            @pl.when(t <= JPW - 2)
            def _(half=half, b=b):
                gather_start(t + 1, half, b)
        return 0

    lax.fori_loop(0, JPW, outer, 0)
    store_wait(JPW - 1, 0, 0)
    store_wait(JPW - 1, 1, 1)


def kernel(weight, token_ids):
    idx_t = token_ids.T.astype(jnp.int32)
    mesh = plsc.VectorSubcoreMesh(core_axis_name="c", subcore_axis_name="s")
    out_p = pl.kernel(
        _emb_body,
        out_type=jax.ShapeDtypeStruct((NJ, KT, NI // 128, 8, 128),
                                      jnp.float32),
        mesh=mesh,
        scratch_types=[
            pltpu.VMEM((JPW, IPW), jnp.int32),
            pltpu.VMEM((2, CH, D_DIM), jnp.float32),
            pltpu.VMEM((2, KT, ITL, 8, 128), jnp.float32),
            pltpu.SemaphoreType.DMA((2,)),
            pltpu.SemaphoreType.DMA((2,)),
        ],
        compiler_params=pltpu.CompilerParams(use_tc_tiling_on_sc=False,
                                             needs_layout_passes=False),
    )(weight, idx_t)
    return out_p.transpose(2, 4, 0, 1, 3).reshape(NI, NJ, D_DIM)


# speedup vs baseline: 1.5593x; 1.5593x over previous
import jax
import jax.numpy as jnp
from jax import lax
from jax.experimental import pallas as pl
from jax.experimental.pallas import tpu as pltpu
from jax.experimental.pallas import tpu_sc as plsc

D_DIM = 32
NI = 4096
NJ = 200
NUM_CORES = 2
NUM_SUBCORES = 16
JG = 8
IG = 4
JPW = NJ // JG
IPW = NI // IG
CH = 512
HALVES = IPW // CH
KT = D_DIM // 8
ITL = CH // 128
SROW = 129


def _emb_body(w_hbm, idx_hbm, out_hbm, idx_v, g_v, s_v, gsem, ssem):
    c = lax.axis_index("c")
    s = lax.axis_index("s")
    w = s * NUM_CORES + c
    jg = w // IG
    ig = w % IG
    j0 = jg * JPW
    it_base = ig * (IPW // 128)

    pltpu.sync_copy(idx_hbm.at[pl.ds(j0, JPW), pl.ds(ig * IPW, IPW)], idx_v)

    iota = lax.iota(jnp.int32, 16)
    ktv0 = iota // 8
    krv0 = iota % 8
    ktv1 = (iota + 16) // 8
    krv1 = (iota + 16) % 8

    def gather_start(jl, half, b):
        pltpu.async_copy(w_hbm.at[idx_v.at[jl, pl.ds(half * CH, CH)]],
                         g_v.at[b], gsem.at[b])

    def gather_wait(jl, half, b):
        pltpu.make_async_copy(w_hbm.at[idx_v.at[jl, pl.ds(half * CH, CH)]],
                              g_v.at[b], gsem.at[b]).wait()

    def transpose(b):
        gv = g_v.at[b]
        sv = s_v.at[b]
        for itl in range(ITL):
            itlv = jnp.full((16,), itl, jnp.int32)

            def ilbody(il0, _, itlv=itlv, itl=itl):
                for u in range(4):
                    il = il0 * 4 + u
                    r = itl * 128 + il
                    ilv = itlv - itl + il
                    plsc.store_scatter(sv, [ktv0, itlv, krv0, ilv],
                                       gv[r, pl.ds(0, 16)])
                    plsc.store_scatter(sv, [ktv1, itlv, krv1, ilv],
                                       gv[r, pl.ds(16, 16)])
                return 0

            lax.fori_loop(0, 32, ilbody, 0)

    def store_start(jl, half, b):
        for kt in range(KT):
            pltpu.async_copy(
                s_v.at[b, kt, :, :, pl.ds(0, 128)],
                out_hbm.at[j0 + jl, kt, pl.ds(it_base + half * ITL, ITL)],
                ssem.at[b])

    def store_wait(jl, half, b):
        for kt in range(KT):
            pltpu.make_async_copy(
                s_v.at[b, kt, :, :, pl.ds(0, 128)],
                out_hbm.at[j0 + jl, kt, pl.ds(it_base + half * ITL, ITL)],
                ssem.at[b]).wait()

    gather_start(0, 0, 0)
    gather_start(0, 1, 1)

    def outer(t, _):
        for half, b in ((0, 0), (1, 1)):
            gather_wait(t, half, b)

            @pl.when(t >= 1)
            def _(half=half, b=b):
                store_wait(t - 1, half, b)

            transpose(b)
            store_start(t, half, b)

            @pl.when(t <= JPW - 2)
            def _(half=half, b=b):
                gather_start(t + 1, half, b)
        return 0

    lax.fori_loop(0, JPW, outer, 0)
    store_wait(JPW - 1, 0, 0)
    store_wait(JPW - 1, 1, 1)


def kernel(weight, token_ids):
    idx_t = token_ids.T.astype(jnp.int32)
    mesh = plsc.VectorSubcoreMesh(core_axis_name="c", subcore_axis_name="s")
    out_p = pl.kernel(
        _emb_body,
        out_type=jax.ShapeDtypeStruct((NJ, KT, NI // 128, 8, 128),
                                      jnp.float32),
        mesh=mesh,
        scratch_types=[
            pltpu.VMEM((JPW, IPW), jnp.int32),
            pltpu.VMEM((2, CH, D_DIM), jnp.float32),
            pltpu.VMEM((2, KT, ITL, 8, SROW), jnp.float32),
            pltpu.SemaphoreType.DMA((2,)),
            pltpu.SemaphoreType.DMA((2,)),
        ],
        compiler_params=pltpu.CompilerParams(use_tc_tiling_on_sc=False,
                                             needs_layout_passes=False),
    )(weight, idx_t)
    return out_p.transpose(2, 4, 0, 1, 3).reshape(NI, NJ, D_DIM)


# speedup vs baseline: 1.5822x; 1.0147x over previous
import jax
import jax.numpy as jnp
from jax import lax
from jax.experimental import pallas as pl
from jax.experimental.pallas import tpu as pltpu
from jax.experimental.pallas import tpu_sc as plsc

D_DIM = 32
NI = 4096
NJ = 200
NUM_CORES = 2
NUM_SUBCORES = 16
JG = 8
IG = 4
JPW = NJ // JG
IPW = NI // IG
CH = 512
HALVES = IPW // CH
ITEMS = JPW * HALVES
KT = D_DIM // 8
ITL = CH // 128
SROW = 129
ITLP = ITL + 1
TPR = 4


def _emb_body(w_hbm, idx_hbm, out_hbm, idx_v, g_v, s_v, gsem, ssem):
    c = lax.axis_index("c")
    s = lax.axis_index("s")
    w = s * NUM_CORES + c
    jg = w // IG
    ig = w % IG
    j0 = jg * JPW
    it_base = ig * (IPW // 128)

    pltpu.sync_copy(idx_hbm.at[pl.ds(j0, JPW), pl.ds(ig * IPW, IPW)], idx_v)

    iota = lax.iota(jnp.int32, 16)
    ktv0 = iota // 8
    krv0 = iota % 8
    ktv1 = (iota + 16) // 8
    krv1 = (iota + 16) % 8
    zerov = iota - iota

    def start_item(g, b):
        jl = g // HALVES
        half = g % HALVES
        pltpu.async_copy(w_hbm.at[idx_v.at[jl, pl.ds(half * CH, CH)]],
                         g_v.at[b], gsem.at[b])

    def wait_item(g, b):
        jl = g // HALVES
        half = g % HALVES
        pltpu.make_async_copy(w_hbm.at[idx_v.at[jl, pl.ds(half * CH, CH)]],
                              g_v.at[b], gsem.at[b]).wait()

    def transpose(b):
        gv = g_v.at[b]
        sv = s_v.at[b]
        for itl in range(ITL):
            itlv = zerov + itl

            def ilbody(z, _, itlv=itlv, itl=itl):
                for u in range(4):
                    il = z * 4 + u
                    r = itl * 128 + il
                    ilv = zerov + il
                    plsc.store_scatter(sv, [ktv0, itlv, krv0, ilv],
                                       gv[r, pl.ds(0, 16)])
                    plsc.store_scatter(sv, [ktv1, itlv, krv1, ilv],
                                       gv[r, pl.ds(16, 16)])
                return 0

            lax.fori_loop(0, 32, ilbody, 0)

    def store_start(g, b):
        jl = g // HALVES
        half = g % HALVES
        for kt in range(KT):
            for itl in range(ITL):
                pltpu.async_copy(
                    s_v.at[b, kt, itl, :, pl.ds(0, 128)],
                    out_hbm.at[j0 + jl, kt, it_base + half * ITL + itl],
                    ssem.at[b])

    def store_wait(g, b):
        jl = g // HALVES
        half = g % HALVES
        for kt in range(KT):
            for itl in range(ITL):
                pltpu.make_async_copy(
                    s_v.at[b, kt, itl, :, pl.ds(0, 128)],
                    out_hbm.at[j0 + jl, kt, it_base + half * ITL + itl],
                    ssem.at[b]).wait()

    start_item(0, 0)
    start_item(1, 1)

    def outer(t, _):
        for par, b in ((0, 0), (1, 1)):
            g = 2 * t + par
            wait_item(g, b)

            @pl.when(t >= 1)
            def _(g=g, b=b):
                store_wait(g - 2, b)

            transpose(b)
            store_start(g, b)

            @pl.when(t <= ITEMS // 2 - 2)
            def _(g=g, b=b):
                start_item(g + 2, b)
        return 0

    lax.fori_loop(0, ITEMS // 2, outer, 0)
    store_wait(ITEMS - 2, 0)
    store_wait(ITEMS - 1, 1)


def kernel(weight, token_ids):
    idx_t = token_ids.T.astype(jnp.int32)
    mesh = plsc.VectorSubcoreMesh(core_axis_name="c", subcore_axis_name="s")
    out_p = pl.kernel(
        _emb_body,
        out_type=jax.ShapeDtypeStruct((NJ, KT, NI // 128, 8, 128),
                                      jnp.float32),
        mesh=mesh,
        scratch_types=[
            pltpu.VMEM((JPW, IPW), jnp.int32),
            pltpu.VMEM((2, CH, D_DIM), jnp.float32),
            pltpu.VMEM((2, KT, ITLP, 8, SROW), jnp.float32),
            pltpu.SemaphoreType.DMA((2,)),
            pltpu.SemaphoreType.DMA((2,)),
        ],
        compiler_params=pltpu.CompilerParams(use_tc_tiling_on_sc=False,
                                             needs_layout_passes=False),
    )(weight, idx_t)
    return out_p.transpose(2, 4, 0, 1, 3).reshape(NI, NJ, D_DIM)
